# Initial kernel scaffold; baseline (speedup 1.0000x reference)
import jax
import jax.numpy as jnp
from jax.experimental import pallas as pl


def kernel(x, edge_index, W_xz, b_xz, W_hz, b_hz, W_xr, b_xr, W_hr, b_hr, W_xh, b_xh, W_hh, b_hh, fc_w, fc_b):
    raise NotImplementedError("write your pallas kernel here")


# speedup vs baseline: 26.9927x; 26.9927x over previous
import functools

import jax
import jax.numpy as jnp
from jax import lax
from jax.experimental import pallas as pl
from jax.experimental.pallas import tpu as pltpu
from jax.experimental.pallas import tpu_sc as plsc

NC = 2
NS = 16
C = 128
MAC = 8
GROUP = 2
RING = 4
BLK = 1024
F48 = 48


def _node_pad(n):
  r = ((n + 1 + BLK - 1) // BLK) * BLK
  assert r % (NS * NC * 8) == 0
  return r


def _deg_groups(src2_ref, ones_v, deg_s, sidx, esems, asems, row0, ngroups):
  def body(g, carry):
    base = row0 + g * (GROUP * MAC)
    pltpu.make_async_copy(src2_ref.at[pl.ds(row0, MAC)], sidx[0], esems[0]).wait()
    pltpu.async_copy(src2_ref.at[pl.ds(base + MAC, MAC)], sidx[1], esems[1])
    waited_second = False
    for st in range(GROUP * MAC):
      slot, j = divmod(st, MAC)
      if slot == 1 and not waited_second:
        pltpu.make_async_copy(
            src2_ref.at[pl.ds(row0, MAC)], sidx[1], esems[1]).wait()
        waited_second = True
      r = st % RING
      if st >= RING:
        ps, pj = divmod(st - RING, MAC)
        pltpu.make_async_copy(
            ones_v, deg_s.at[sidx[ps].at[jnp.int32(pj)]], asems[r]).wait()
      pltpu.async_copy(ones_v, deg_s.at[sidx[slot].at[jnp.int32(j)]], asems[r],
                       add=True)
    for st in range(GROUP * MAC - RING, GROUP * MAC):
      slot, j = divmod(st, MAC)
      pltpu.make_async_copy(
          ones_v, deg_s.at[sidx[slot].at[jnp.int32(j)]], asems[st % RING]).wait()

    @pl.when(g + 1 < ngroups)
    def _():
      pltpu.async_copy(
          src2_ref.at[pl.ds(base + GROUP * MAC, MAC)], sidx[0], esems[0])

    return carry

  lax.fori_loop(jnp.int32(0), jnp.int32(ngroups), body, jnp.int32(0))


def _make_deg_kernel(e_pad, r_rows):
  rpt = r_rows // NS
  rows_per_tile = e_pad // (NC * NS) // C
  ngroups = rows_per_tile // (GROUP * MAC)
  mesh = plsc.VectorSubcoreMesh(core_axis_name="c", subcore_axis_name="s")

  @functools.partial(
      pl.kernel,
      out_type=jax.ShapeDtypeStruct((2 * r_rows, 16), jnp.float32),
      mesh=mesh,
      compiler_params=pltpu.CompilerParams(use_tc_tiling_on_sc=False),
      name="sc_deg",
      scratch_types=[
          pltpu.VMEM_SHARED((r_rows, 16), jnp.float32),
          pltpu.VMEM((MAC, C), jnp.int32),
          pltpu.VMEM((MAC, C), jnp.int32),
          pltpu.VMEM((C, 16), jnp.float32),
          pltpu.SemaphoreType.DMA(()),
          pltpu.SemaphoreType.DMA(()),
          pltpu.SemaphoreType.DMA(()),
          pltpu.SemaphoreType.DMA(()),
          pltpu.SemaphoreType.DMA(()),
          pltpu.SemaphoreType.DMA(()),
      ],
  )
  def deg_kernel(src2_ref, ones_ref, zeros_ref, out_ref,
                 deg_s, sidx0, sidx1, ones_v, e0, e1, a0, a1, a2, a3):
    c = lax.axis_index("c")
    s = lax.axis_index("s")
    tid_row = s * rpt
    pltpu.sync_copy(zeros_ref, deg_s.at[pl.ds(tid_row, rpt)])
    pltpu.sync_copy(ones_ref, ones_v)
    plsc.subcore_barrier()
    row0 = (c * NS + s) * rows_per_tile
    pltpu.async_copy(src2_ref.at[pl.ds(row0, MAC)], sidx0, e0)
    _deg_groups(src2_ref, ones_v, deg_s, [sidx0, sidx1], [e0, e1],
                [a0, a1, a2, a3], row0, ngroups)
    plsc.subcore_barrier()
    pltpu.sync_copy(deg_s.at[pl.ds(tid_row, rpt)],
                    out_ref.at[pl.ds(c * r_rows + tid_row, rpt)])

  return deg_kernel


def _make_prep_kernel(r_rows):
  rpt32 = r_rows // (NC * NS)
  q = 448
  nch = rpt32 // q
  assert nch * q == rpt32
  mesh = plsc.VectorSubcoreMesh(core_axis_name="c", subcore_axis_name="s")

  @functools.partial(
      pl.kernel,
      out_type=[jax.ShapeDtypeStruct((r_rows, 16), jnp.float32)
                for _ in range(4)],
      mesh=mesh,
      compiler_params=pltpu.CompilerParams(use_tc_tiling_on_sc=False),
      name="sc_prep",
      scratch_types=[
          pltpu.VMEM((q, 16), jnp.float32),
          pltpu.VMEM((q, 16), jnp.float32),
          pltpu.VMEM((q, F48), jnp.float32),
          pltpu.VMEM((q, 16), jnp.float32),
          pltpu.VMEM((q, 16), jnp.float32),
          pltpu.VMEM((q, 16), jnp.float32),
          pltpu.VMEM((q, 16), jnp.float32),
      ],
  )
  def prep_kernel(degs_ref, x_ref, y0_out, y1_out, y2_out, dv_out,
                  d0b, d1b, xb, dvb, y0b, y1b, y2b):
    c = lax.axis_index("c")
    s = lax.axis_index("s")
    w = s * NC + c
    base = w * rpt32

    def chunk(ch, carry):
      row = base + ch * q
      pltpu.sync_copy(degs_ref.at[pl.ds(row, q)], d0b)
      pltpu.sync_copy(degs_ref.at[pl.ds(r_rows + row, q)], d1b)
      pltpu.sync_copy(x_ref.at[pl.ds(row, q)], xb)

      def body(i, cc):
        d = d0b[i, :] + d1b[i, :]
        t = lax.bitcast_convert_type(d, jnp.int32)
        t = jnp.int32(0x5F3759DF) - lax.shift_right_logical(t, jnp.int32(1))
        z = lax.bitcast_convert_type(t, jnp.float32)
        for _ in range(3):
          z = z * (1.5 - 0.5 * d * z * z)
        z = jnp.where(d > 0.0, z, 0.0)
        dvb[i, :] = z
        y0b[i, :] = xb[i, pl.ds(0, 16)] * z
        y1b[i, :] = xb[i, pl.ds(16, 16)] * z
        y2b[i, :] = xb[i, pl.ds(32, 16)] * z
        return cc

      lax.fori_loop(0, q, body, jnp.int32(0), unroll=4)
      pltpu.sync_copy(dvb, dv_out.at[pl.ds(row, q)])
      pltpu.sync_copy(y0b, y0_out.at[pl.ds(row, q)])
      pltpu.sync_copy(y1b, y1_out.at[pl.ds(row, q)])
      pltpu.sync_copy(y2b, y2_out.at[pl.ds(row, q)])
      return carry

    lax.fori_loop(jnp.int32(0), jnp.int32(nch), chunk, jnp.int32(0))

  return prep_kernel


def _agg_pass(src2_ref, dst2_ref, tab_ref, agg_s, sidx, didx, rows, esems,
              gsems, asems, row0, ngroups):
  def body(g, carry):
    base = row0 + g * (GROUP * MAC)
    pltpu.make_async_copy(src2_ref.at[pl.ds(row0, MAC)], sidx[0], esems[0]).wait()
    pltpu.make_async_copy(dst2_ref.at[pl.ds(row0, MAC)], didx[0], esems[0]).wait()
    pltpu.async_copy(src2_ref.at[pl.ds(base + MAC, MAC)], sidx[1], esems[1])
    pltpu.async_copy(dst2_ref.at[pl.ds(base + MAC, MAC)], didx[1], esems[1])
    waited_second = False
    nst = GROUP * MAC
    for st in range(nst + 2):
      if st < nst:
        slot, j = divmod(st, MAC)
        if slot == 1 and not waited_second:
          pltpu.make_async_copy(
              src2_ref.at[pl.ds(row0, MAC)], sidx[1], esems[1]).wait()
          pltpu.make_async_copy(
              dst2_ref.at[pl.ds(row0, MAC)], didx[1], esems[1]).wait()
          waited_second = True
        r = st % RING
        if st >= RING:
          ps, pj = divmod(st - RING, MAC)
          pltpu.make_async_copy(
              rows[r], agg_s.at[didx[ps].at[jnp.int32(pj)]], asems[r]).wait()
        pltpu.async_copy(tab_ref.at[sidx[slot].at[jnp.int32(j)]], rows[r],
                         gsems[r])
      if st >= 2:
        st2 = st - 2
        s2, j2 = divmod(st2, MAC)
        r2 = st2 % RING
        pltpu.make_async_copy(
            tab_ref.at[sidx[s2].at[jnp.int32(j2)]], rows[r2], gsems[r2]).wait()
        pltpu.async_copy(rows[r2], agg_s.at[didx[s2].at[jnp.int32(j2)]],
                         asems[r2], add=True)
    for st in range(nst - RING, nst):
      slot, j = divmod(st, MAC)
      pltpu.make_async_copy(
          rows[st % RING], agg_s.at[didx[slot].at[jnp.int32(j)]],
          asems[st % RING]).wait()

    @pl.when(g + 1 < ngroups)
    def _():
      pltpu.async_copy(src2_ref.at[pl.ds(base + GROUP * MAC, MAC)], sidx[0],
                       esems[0])
      pltpu.async_copy(dst2_ref.at[pl.ds(base + GROUP * MAC, MAC)], didx[0],
                       esems[0])

    return carry

  lax.fori_loop(jnp.int32(0), jnp.int32(ngroups), body, jnp.int32(0))


def _make_agg_kernel(e_pad, r_rows):
  rpt = r_rows // NS
  q = 196
  nch = rpt // q
  assert nch * q == rpt
  erows = e_pad // C
  full_rpt = erows // NS
  half_rpt = erows // (2 * NS)
  ngroups_full = full_rpt // (GROUP * MAC)
  ngroups_half = half_rpt // (GROUP * MAC)
  mesh = plsc.VectorSubcoreMesh(core_axis_name="c", subcore_axis_name="s")

  @functools.partial(
      pl.kernel,
      out_type=jax.ShapeDtypeStruct((r_rows, 128), jnp.float32),
      mesh=mesh,
      compiler_params=pltpu.CompilerParams(use_tc_tiling_on_sc=False),
      name="sc_agg",
      scratch_types=(
          [pltpu.VMEM_SHARED((r_rows, 16), jnp.float32)]
          + [pltpu.VMEM((MAC, C), jnp.int32) for _ in range(4)]
          + [pltpu.VMEM((C, 16), jnp.float32) for _ in range(RING)]
          + [pltpu.VMEM((q, 16), jnp.float32) for _ in range(3)]
          + [pltpu.SemaphoreType.DMA(()) for _ in range(2 + 2 * RING)]
      ),
  )
  def agg_kernel(src2_ref, dst2_ref, y0_ref, y1_ref, y2_ref, dv_ref,
                 zeros_ref, out_ref, agg_s, *rest):
    sidx = list(rest[0:2])
    didx = list(rest[2:4])
    rows = list(rest[4:4 + RING])
    dvb, ab, zb = rest[4 + RING:7 + RING]
    esems = list(rest[7 + RING:9 + RING])
    gsems = list(rest[9 + RING:9 + 2 * RING])
    asems = list(rest[9 + 2 * RING:9 + 3 * RING])
    c = lax.axis_index("c")
    s = lax.axis_index("s")
    tid_row = s * rpt

    def scaled_dump(col0):
      def chunk(ch, carry):
        row = tid_row + ch * q
        pltpu.sync_copy(dv_ref.at[pl.ds(row, q)], dvb)
        pltpu.sync_copy(agg_s.at[pl.ds(row, q)], ab)

        def body(i, cc):
          ab[i, :] = ab[i, :] * (0.0 - dvb[i, :])
          return cc

        lax.fori_loop(0, q, body, jnp.int32(0), unroll=8)
        pltpu.sync_copy(ab, out_ref.at[pl.ds(row, q), pl.ds(col0, 16)])
        return carry

      lax.fori_loop(jnp.int32(0), jnp.int32(nch), chunk, jnp.int32(0))

    def run_phase(tab_ref, row0, ngroups, col0):
      pltpu.sync_copy(zeros_ref, agg_s.at[pl.ds(tid_row, rpt)])
      plsc.subcore_barrier()
      pltpu.async_copy(src2_ref.at[pl.ds(row0, MAC)], sidx[0], esems[0])
      pltpu.async_copy(dst2_ref.at[pl.ds(row0, MAC)], didx[0], esems[0])
      _agg_pass(src2_ref, dst2_ref, tab_ref, agg_s, sidx, didx, rows, esems,
                gsems, asems, row0, ngroups)
      plsc.subcore_barrier()
      scaled_dump(col0)
      plsc.subcore_barrier()

    row0_full = s * full_rpt

    @pl.when(c == 0)
    def _():
      run_phase(y0_ref, row0_full, ngroups_full, jnp.int32(0))

    @pl.when(c == 1)
    def _():
      run_phase(y2_ref, row0_full, ngroups_full, jnp.int32(32))

    row0_half = c * (erows // 2) + s * half_rpt
    run_phase(y1_ref, row0_half, ngroups_half, 16 + c * 32)

    pltpu.sync_copy(zeros_ref.at[pl.ds(jnp.int32(0), q)], zb)
    def zfill(ch, carry):
      row = tid_row + ch * q
      zcol = 64 + c * 32
      pltpu.sync_copy(zb, out_ref.at[pl.ds(row, q), pl.ds(zcol, 16)])
      pltpu.sync_copy(zb, out_ref.at[pl.ds(row, q), pl.ds(zcol + 16, 16)])
      return carry
    lax.fori_loop(jnp.int32(0), jnp.int32(nch), zfill, jnp.int32(0))

  return agg_kernel


def _gru_body(x_ref, lh_ref, wz_ref, wh_ref, bz_ref, bh_ref, fw_ref, fb_ref,
              o_ref):
  u = jnp.concatenate([x_ref[:, :], lh_ref[:, :]], axis=1)
  zp = jnp.dot(u, wz_ref[:, :], preferred_element_type=jnp.float32) + bz_ref[:, :]
  hp = jnp.dot(u, wh_ref[:, :], preferred_element_type=jnp.float32) + bh_ref[:, :]
  z = jax.nn.sigmoid(zp)
  ht = jnp.tanh(hp)
  hr = jnp.maximum((1.0 - z) * ht, 0.0)
  o = jnp.dot(hr, fw_ref[:, :], preferred_element_type=jnp.float32) + fb_ref[:, :]
  o_ref[:, :] = jnp.maximum(o, 0.0)


def _make_gru(r_rows, f_hid):
  rb = r_rows // BLK
  return pl.pallas_call(
      _gru_body,
      name="tc_gru",
      grid=(rb,),
      in_specs=[
          pl.BlockSpec((BLK, F48), lambda i: (i, jnp.int32(0))),
          pl.BlockSpec((BLK, 128), lambda i: (i, jnp.int32(0))),
          pl.BlockSpec((F48 + 128, f_hid),
                       lambda i: (jnp.int32(0), jnp.int32(0))),
          pl.BlockSpec((F48 + 128, f_hid),
                       lambda i: (jnp.int32(0), jnp.int32(0))),
          pl.BlockSpec((1, f_hid), lambda i: (jnp.int32(0), jnp.int32(0))),
          pl.BlockSpec((1, f_hid), lambda i: (jnp.int32(0), jnp.int32(0))),
          pl.BlockSpec((f_hid, F48), lambda i: (jnp.int32(0), jnp.int32(0))),
          pl.BlockSpec((1, F48), lambda i: (jnp.int32(0), jnp.int32(0))),
      ],
      out_specs=pl.BlockSpec((BLK, F48), lambda i: (i, jnp.int32(0))),
      out_shape=jax.ShapeDtypeStruct((r_rows, F48), jnp.float32),
  )


def kernel(x, edge_index, W_xz, b_xz, W_hz, b_hz, W_xr, b_xr, W_hr, b_hr,
           W_xh, b_xh, W_hh, b_hh, fc_w, fc_b):
  f32 = jnp.float32
  n, f_in = x.shape
  f_hid = W_xz.shape[2]
  e = edge_index.shape[1]
  r_rows = _node_pad(n)
  dum = n

  quantum = NC * NS * C * MAC * GROUP
  e_pad = ((e + quantum - 1) // quantum) * quantum
  src = edge_index[0].astype(jnp.int32)
  dst = edge_index[1].astype(jnp.int32)
  padv = jnp.full((e_pad - e,), dum, jnp.int32)
  src2 = jnp.concatenate([src, padv]).reshape(e_pad // C, C)
  dst2 = jnp.concatenate([dst, padv]).reshape(e_pad // C, C)

  zeros_in = jnp.zeros((r_rows // NS, 16), f32)
  ones_in = jnp.ones((C, 16), f32)

  degs = _make_deg_kernel(e_pad, r_rows)(src2, ones_in, zeros_in)

  x48 = jnp.pad(x.astype(f32), ((0, r_rows - n), (0, F48 - f_in)))
  y0, y1, y2, dv = _make_prep_kernel(r_rows)(degs, x48)

  lh = _make_agg_kernel(e_pad, r_rows)(src2, dst2, y0, y1, y2, dv, zeros_in)

  def padw(w, rows):
    return jnp.pad(w.astype(f32), ((0, rows - w.shape[0]), (0, 0)))

  def wcat(wpair):
    w0 = padw(wpair[0], F48)
    w1 = padw(wpair[1], F48)
    w1b = w1[16:32]
    z64 = jnp.zeros((64, f_hid), f32)
    return jnp.concatenate([w0, w1, w1b, z64], axis=0)

  wz = wcat(W_xz)
  wh = wcat(W_xh)
  bz = (b_xz + b_hz).astype(f32).reshape(1, f_hid)
  bh = (b_xh + b_hh).astype(f32).reshape(1, f_hid)
  fw = jnp.pad(fc_w.astype(f32), ((0, 0), (0, F48 - f_in)))
  fb = jnp.pad(fc_b.astype(f32), (0, F48 - f_in)).reshape(1, F48)

  out48 = _make_gru(r_rows, f_hid)(x48, lh, wz, wh, bz, bh, fw, fb)
  return out48[:n, :f_in]
